# baseline (speedup 1.0000x reference)
import functools

import jax
import jax.numpy as jnp
from jax import lax
from jax.experimental import pallas as pl
from jax.experimental.pallas import tpu as pltpu
from jax.experimental.pallas import tpu_sc as plsc

_NC = 2
_NS = 16
_NW = _NC * _NS
_L = 16
_B = 128
_CB = 16
_CP = _B * _CB
_CW = 2 * _CP


def _lj_body(n_chunks, pairs_hbm, tab_hbm, consts_hbm, out_hbm,
             idx_v, rows_v, consts_v, acc_v, sem):
    cid = lax.axis_index("c")
    sid = lax.axis_index("s")
    wid = sid * _NC + cid

    pltpu.sync_copy(consts_hbm, consts_v)
    cv0 = consts_v[pl.ds(0, _L)]
    cv1 = consts_v[pl.ds(8, _L)]

    def cget(i):
        return cv0[i] if i < _L else cv1[i - 8]

    bi = [cget(k) for k in range(9)]
    bx = [cget(9 + k) for k in range(9)]
    cut2 = cget(18)

    lane1 = lax.iota(jnp.int32, _L)
    zero16 = jnp.zeros((_L,), jnp.int32)

    max_chunks = (n_chunks - 1) // _NW + 1

    n_runs = 2 * _CB

    def chunk_body(i, acc):
        g_raw = i * _NW + wid
        valid = g_raw < n_chunks
        g = jnp.where(valid, g_raw, 0)
        pltpu.sync_copy(pairs_hbm.at[pl.ds(g * n_runs, n_runs), :], idx_v)
        for u in range(n_runs):
            pltpu.async_copy(tab_hbm.at[idx_v.at[u]], rows_v.at[u], sem)
        for u in range(n_runs):
            pltpu.make_async_copy(tab_hbm.at[idx_v.at[u]], rows_v.at[u],
                                  sem).wait()

        def inner(j, acc):
            jb = j // 8
            js = j - jb * 8
            run0 = 2 * jb
            pos = js * 16 + lane1
            f = [plsc.load_gather(rows_v, [zero16 + run0 + e, pos, zero16 + k])
                 for e in (0, 1) for k in range(5)]
            x0, y0, z0, s0, e0, x1, y1, z1, s1, e1 = f

            def bfr(v):
                u = plsc.bitcast(v, jnp.int32)
                u = (u + 0x7FFF + ((u >> 16) & 1)) & -65536
                return plsc.bitcast(u, jnp.float32)

            dx = bfr(x0 - x1)
            dy = bfr(y0 - y1)
            dz = bfr(z0 - z1)
            sx = dx * bi[0] + dy * bi[3] + dz * bi[6]
            sy = dx * bi[1] + dy * bi[4] + dz * bi[7]
            sz = dx * bi[2] + dy * bi[5] + dz * bi[8]

            def wrap(s):
                y = s + 0.5
                t = y.astype(jnp.int32).astype(jnp.float32)
                fl = jnp.where(t > y, t - 1.0, t)
                return s - fl

            wx = bfr(wrap(sx))
            wy = bfr(wrap(sy))
            wz = bfr(wrap(sz))
            px = wx * bx[0] + wy * bx[3] + wz * bx[6]
            py = wx * bx[1] + wy * bx[4] + wz * bx[7]
            pz = wx * bx[2] + wy * bx[5] + wz * bx[8]
            r2 = px * px + py * py + pz * pz
            sig = (s0 + s1) * 0.5
            inv = 1.0 / r2
            inv = inv * (2.0 - r2 * inv)
            inv = inv * (2.0 - r2 * inv)
            q = (sig * sig) * inv
            t3 = q * q * q
            ene = (4.0 * (e0 * e1)) * (t3 * (t3 - 1.0))
            return acc + jnp.where(r2 <= cut2, ene, 0.0)

        chunk_acc = lax.fori_loop(0, _CP // _L, inner,
                                  jnp.zeros((_L,), jnp.float32))
        return acc + jnp.where(valid, chunk_acc, 0.0)

    acc = lax.fori_loop(0, max_chunks, chunk_body,
                        jnp.zeros((_L,), jnp.float32))
    acc_v[...] = acc
    pltpu.sync_copy(acc_v, out_hbm.at[wid])


@functools.partial(jax.jit, static_argnums=(3,))
def _lj_launch(pairs_lin, tab, consts, n_chunks):
    mesh = plsc.VectorSubcoreMesh(core_axis_name="c", subcore_axis_name="s")
    body = functools.partial(_lj_body, n_chunks)
    out = pl.kernel(
        body,
        out_type=jax.ShapeDtypeStruct((_NW, _L), jnp.float32),
        mesh=mesh,
        compiler_params=pltpu.CompilerParams(
            needs_layout_passes=False, use_tc_tiling_on_sc=False),
        scratch_types=[
            pltpu.VMEM((2 * _CB, _B), jnp.int32),
            pltpu.VMEM((2 * _CB, _B, 8), jnp.float32),
            pltpu.VMEM((24,), jnp.float32),
            pltpu.VMEM((_L,), jnp.float32),
            pltpu.SemaphoreType.DMA,
        ],
    )(pairs_lin, tab, consts)
    return jnp.sum(out)


def kernel(coords, pairs, box, sigma, epsilon, cutoff):
    n = coords.shape[0]
    p = pairs.shape[0]
    nb = p // _B
    assert p % _B == 0 and nb % _CB == 0, p
    box = box.astype(jnp.float32)
    box_inv = jnp.linalg.inv(box)
    tab = jnp.concatenate(
        [coords.astype(jnp.float32),
         sigma.astype(jnp.float32)[:, None],
         jnp.sqrt(epsilon.astype(jnp.float32))[:, None],
         jnp.zeros((n, 3), jnp.float32)], axis=1)
    cut2 = (jnp.asarray(cutoff, jnp.float32) ** 2).reshape(1)
    def _rne_bf(x):
        u = lax.bitcast_convert_type(x, jnp.int32)
        u = (u + 0x7FFF + ((u >> 16) & 1)) & -65536
        return lax.bitcast_convert_type(u, jnp.float32)

    bi_bf = _rne_bf(box_inv)
    bx_bf = _rne_bf(box)
    consts = jnp.concatenate(
        [bi_bf.reshape(-1), bx_bf.reshape(-1), cut2,
         jnp.zeros((5,), jnp.float32)]).astype(jnp.float32)
    pairs_lin = (pairs.astype(jnp.int32).T
                 .reshape(2, nb, _B).transpose(1, 0, 2).reshape(-1))
    return _lj_launch(pairs_lin.reshape(2 * nb, _B), tab, consts, nb // _CB)


# speedup vs baseline: 1.0067x; 1.0067x over previous
import functools

import jax
import jax.numpy as jnp
from jax import lax
from jax.experimental import pallas as pl
from jax.experimental.pallas import tpu as pltpu
from jax.experimental.pallas import tpu_sc as plsc

_NC = 2
_NS = 16
_NW = _NC * _NS
_L = 16
_B = 128
_CB = 16
_CP = _B * _CB
_CW = 2 * _CP


def _lj_body(n_chunks, pairs_hbm, tab_hbm, consts_hbm, out_hbm,
             idx_v, rows_v, consts_v, acc_v, sem):
    cid = lax.axis_index("c")
    sid = lax.axis_index("s")
    wid = sid * _NC + cid

    pltpu.sync_copy(consts_hbm, consts_v)
    cv0 = consts_v[pl.ds(0, _L)]
    cv1 = consts_v[pl.ds(8, _L)]

    def cget(i):
        return cv0[i] if i < _L else cv1[i - 8]

    bi = [cget(k) for k in range(9)]
    bx = [cget(9 + k) for k in range(9)]
    cut2 = cget(18)

    lane1 = lax.iota(jnp.int32, _L)
    zero16 = jnp.zeros((_L,), jnp.int32)

    max_chunks = (n_chunks - 1) // _NW + 1

    def chunk_body(i, acc):
        g_raw = i * _NW + wid
        valid = g_raw < n_chunks
        g = jnp.where(valid, g_raw, 0)
        pltpu.sync_copy(pairs_hbm.at[pl.ds(g * _CW, _CW)], idx_v)
        pltpu.async_copy(tab_hbm.at[idx_v], rows_v, sem).wait()

        def inner(j, acc):
            jb = j // 8
            js = j - jb * 8
            r0 = jb * 256 + js * 16 + lane1
            r1 = r0 + 128
            f = [plsc.load_gather(rows_v, [r, zero16 + k])
                 for r in (r0, r1) for k in range(5)]
            x0, y0, z0, s0, e0, x1, y1, z1, s1, e1 = f

            def bfr(v):
                u = plsc.bitcast(v, jnp.int32)
                u = (u + 0x7FFF + ((u >> 16) & 1)) & -65536
                return plsc.bitcast(u, jnp.float32)

            dx = bfr(x0 - x1)
            dy = bfr(y0 - y1)
            dz = bfr(z0 - z1)
            sx = dx * bi[0] + dy * bi[3] + dz * bi[6]
            sy = dx * bi[1] + dy * bi[4] + dz * bi[7]
            sz = dx * bi[2] + dy * bi[5] + dz * bi[8]

            def wrap(s):
                y = s + 0.5
                t = y.astype(jnp.int32).astype(jnp.float32)
                fl = jnp.where(t > y, t - 1.0, t)
                return s - fl

            wx = bfr(wrap(sx))
            wy = bfr(wrap(sy))
            wz = bfr(wrap(sz))
            px = wx * bx[0] + wy * bx[3] + wz * bx[6]
            py = wx * bx[1] + wy * bx[4] + wz * bx[7]
            pz = wx * bx[2] + wy * bx[5] + wz * bx[8]
            r2 = px * px + py * py + pz * pz
            sig = (s0 + s1) * 0.5
            inv = 1.0 / r2
            inv = inv * (2.0 - r2 * inv)
            inv = inv * (2.0 - r2 * inv)
            q = (sig * sig) * inv
            t3 = q * q * q
            ene = (4.0 * (e0 * e1)) * (t3 * (t3 - 1.0))
            return acc + jnp.where(r2 <= cut2, ene, 0.0)

        chunk_acc = lax.fori_loop(0, _CP // _L, inner,
                                  jnp.zeros((_L,), jnp.float32))
        return acc + jnp.where(valid, chunk_acc, 0.0)

    acc = lax.fori_loop(0, max_chunks, chunk_body,
                        jnp.zeros((_L,), jnp.float32))
    acc_v[...] = acc
    pltpu.sync_copy(acc_v, out_hbm.at[wid])


@functools.partial(jax.jit, static_argnums=(3,))
def _lj_launch(pairs_lin, tab, consts, n_chunks):
    mesh = plsc.VectorSubcoreMesh(core_axis_name="c", subcore_axis_name="s")
    body = functools.partial(_lj_body, n_chunks)
    out = pl.kernel(
        body,
        out_type=jax.ShapeDtypeStruct((_NW, _L), jnp.float32),
        mesh=mesh,
        compiler_params=pltpu.CompilerParams(
            needs_layout_passes=False, use_tc_tiling_on_sc=False),
        scratch_types=[
            pltpu.VMEM((_CW,), jnp.int32),
            pltpu.VMEM((_CW, 8), jnp.float32),
            pltpu.VMEM((24,), jnp.float32),
            pltpu.VMEM((_L,), jnp.float32),
            pltpu.SemaphoreType.DMA,
        ],
    )(pairs_lin, tab, consts)
    return jnp.sum(out)


def kernel(coords, pairs, box, sigma, epsilon, cutoff):
    n = coords.shape[0]
    p = pairs.shape[0]
    nb = p // _B
    assert p % _B == 0 and nb % _CB == 0, p
    box = box.astype(jnp.float32)
    box_inv = jnp.linalg.inv(box)
    tab = jnp.concatenate(
        [coords.astype(jnp.float32),
         sigma.astype(jnp.float32)[:, None],
         jnp.sqrt(epsilon.astype(jnp.float32))[:, None],
         jnp.zeros((n, 3), jnp.float32)], axis=1)
    cut2 = (jnp.asarray(cutoff, jnp.float32) ** 2).reshape(1)
    def _rne_bf(x):
        u = lax.bitcast_convert_type(x, jnp.int32)
        u = (u + 0x7FFF + ((u >> 16) & 1)) & -65536
        return lax.bitcast_convert_type(u, jnp.float32)

    bi_bf = _rne_bf(box_inv)
    bx_bf = _rne_bf(box)
    consts = jnp.concatenate(
        [bi_bf.reshape(-1), bx_bf.reshape(-1), cut2,
         jnp.zeros((5,), jnp.float32)]).astype(jnp.float32)
    pairs_lin = (pairs.astype(jnp.int32).T
                 .reshape(2, nb, _B).transpose(1, 0, 2).reshape(-1))
    return _lj_launch(pairs_lin, tab, consts, nb // _CB)


# speedup vs baseline: 1.8087x; 1.7967x over previous
import functools

import jax
import jax.numpy as jnp
from jax import lax
from jax.experimental import pallas as pl
from jax.experimental.pallas import tpu as pltpu
from jax.experimental.pallas import tpu_sc as plsc

_NC = 2
_NS = 16
_NW = _NC * _NS
_L = 16
_B = 128
_CB = 16
_CP = _B * _CB
_CW = 2 * _CP


def _lj_body(n_chunks, pairs_hbm, tab_hbm, consts_hbm, out_hbm,
             idx0_v, idx1_v, rows0_v, rows1_v, consts_v, acc_v,
             sp0, sp1, sg0, sg1):
    idx_b = (idx0_v, idx1_v)
    rows_b = (rows0_v, rows1_v)
    sp_b = (sp0, sp1)
    sg_b = (sg0, sg1)
    cid = lax.axis_index("c")
    sid = lax.axis_index("s")
    wid = sid * _NC + cid

    pltpu.sync_copy(consts_hbm, consts_v)
    cv0 = consts_v[pl.ds(0, _L)]
    cv1 = consts_v[pl.ds(8, _L)]

    def cget(i):
        return cv0[i] if i < _L else cv1[i - 8]

    bi = [cget(k) for k in range(9)]
    bx = [cget(9 + k) for k in range(9)]
    cut2 = cget(18)

    lane1 = lax.iota(jnp.int32, _L)
    zero16 = jnp.zeros((_L,), jnp.int32)

    max_chunks = (n_chunks - 1) // _NW + 1

    def start_pairs(b, k):
        g_raw = k * _NW + wid
        g = jnp.where(g_raw < n_chunks, g_raw, 0)
        pltpu.async_copy(pairs_hbm.at[pl.ds(g * _CW, _CW)], idx_b[b], sp_b[b])

    def wait_pairs(b):
        pltpu.make_async_copy(pairs_hbm.at[pl.ds(0, _CW)], idx_b[b],
                              sp_b[b]).wait()

    def start_gather(b):
        pltpu.async_copy(tab_hbm.at[idx_b[b]], rows_b[b], sg_b[b])

    def wait_gather(b):
        pltpu.make_async_copy(tab_hbm.at[idx_b[b]], rows_b[b], sg_b[b]).wait()

    def compute(b):
        rows_v = rows_b[b]

        def inner(j, acc):
            jb = j // 8
            js = j - jb * 8
            r0 = jb * 256 + js * 16 + lane1
            r1 = r0 + 128
            f = [plsc.load_gather(rows_v, [r, zero16 + k])
                 for r in (r0, r1) for k in range(5)]
            x0, y0, z0, s0, e0, x1, y1, z1, s1, e1 = f

            def bfr(v):
                u = plsc.bitcast(v, jnp.int32)
                u = (u + 0x7FFF + ((u >> 16) & 1)) & -65536
                return plsc.bitcast(u, jnp.float32)

            dx = bfr(x0 - x1)
            dy = bfr(y0 - y1)
            dz = bfr(z0 - z1)
            sx = dx * bi[0] + dy * bi[3] + dz * bi[6]
            sy = dx * bi[1] + dy * bi[4] + dz * bi[7]
            sz = dx * bi[2] + dy * bi[5] + dz * bi[8]

            def wrap(s):
                y = s + 0.5
                t = y.astype(jnp.int32).astype(jnp.float32)
                fl = jnp.where(t > y, t - 1.0, t)
                return s - fl

            wx = bfr(wrap(sx))
            wy = bfr(wrap(sy))
            wz = bfr(wrap(sz))
            px = wx * bx[0] + wy * bx[3] + wz * bx[6]
            py = wx * bx[1] + wy * bx[4] + wz * bx[7]
            pz = wx * bx[2] + wy * bx[5] + wz * bx[8]
            r2 = px * px + py * py + pz * pz
            sig = (s0 + s1) * 0.5
            inv = 1.0 / r2
            inv = inv * (2.0 - r2 * inv)
            inv = inv * (2.0 - r2 * inv)
            q = (sig * sig) * inv
            t3 = q * q * q
            ene = (4.0 * (e0 * e1)) * (t3 * (t3 - 1.0))
            return acc + jnp.where(r2 <= cut2, ene, 0.0)

        return lax.fori_loop(0, _CP // _L, inner,
                             jnp.zeros((_L,), jnp.float32))

    K = max_chunks
    assert K >= 4 and K % 2 == 0, K
    assert (K - 1) * _NW - 1 < n_chunks, (K, n_chunks)
    start_pairs(0, 0)
    start_pairs(1, 1)
    wait_pairs(0)
    start_gather(0)

    def pipe_body(i, acc):
        wait_pairs(1)
        start_gather(1)
        wait_gather(0)
        acc = acc + compute(0)
        start_pairs(0, 2 * i + 2)
        wait_pairs(0)
        start_gather(0)
        wait_gather(1)
        acc = acc + compute(1)
        start_pairs(1, 2 * i + 3)
        return acc

    acc = lax.fori_loop(0, (K - 2) // 2, pipe_body,
                        jnp.zeros((_L,), jnp.float32))
    wait_pairs(1)
    start_gather(1)
    wait_gather(0)
    acc = acc + compute(0)
    wait_gather(1)
    last_valid = ((K - 1) * _NW + wid) < n_chunks
    acc = acc + jnp.where(last_valid, compute(1), 0.0)
    acc_v[...] = acc
    pltpu.sync_copy(acc_v, out_hbm.at[wid])


@functools.partial(jax.jit, static_argnums=(3,))
def _lj_launch(pairs_lin, tab, consts, n_chunks):
    mesh = plsc.VectorSubcoreMesh(core_axis_name="c", subcore_axis_name="s")
    body = functools.partial(_lj_body, n_chunks)
    out = pl.kernel(
        body,
        out_type=jax.ShapeDtypeStruct((_NW, _L), jnp.float32),
        mesh=mesh,
        compiler_params=pltpu.CompilerParams(
            needs_layout_passes=False, use_tc_tiling_on_sc=False),
        scratch_types=[
            pltpu.VMEM((_CW,), jnp.int32),
            pltpu.VMEM((_CW,), jnp.int32),
            pltpu.VMEM((_CW, 8), jnp.float32),
            pltpu.VMEM((_CW, 8), jnp.float32),
            pltpu.VMEM((24,), jnp.float32),
            pltpu.VMEM((_L,), jnp.float32),
            pltpu.SemaphoreType.DMA,
            pltpu.SemaphoreType.DMA,
            pltpu.SemaphoreType.DMA,
            pltpu.SemaphoreType.DMA,
        ],
    )(pairs_lin, tab, consts)
    return jnp.sum(out)


def kernel(coords, pairs, box, sigma, epsilon, cutoff):
    n = coords.shape[0]
    p = pairs.shape[0]
    nb = p // _B
    assert p % _B == 0 and nb % _CB == 0, p
    box = box.astype(jnp.float32)
    box_inv = jnp.linalg.inv(box)
    tab = jnp.concatenate(
        [coords.astype(jnp.float32),
         sigma.astype(jnp.float32)[:, None],
         jnp.sqrt(epsilon.astype(jnp.float32))[:, None],
         jnp.zeros((n, 3), jnp.float32)], axis=1)
    cut2 = (jnp.asarray(cutoff, jnp.float32) ** 2).reshape(1)
    def _rne_bf(x):
        u = lax.bitcast_convert_type(x, jnp.int32)
        u = (u + 0x7FFF + ((u >> 16) & 1)) & -65536
        return lax.bitcast_convert_type(u, jnp.float32)

    bi_bf = _rne_bf(box_inv)
    bx_bf = _rne_bf(box)
    consts = jnp.concatenate(
        [bi_bf.reshape(-1), bx_bf.reshape(-1), cut2,
         jnp.zeros((5,), jnp.float32)]).astype(jnp.float32)
    pairs_lin = (pairs.astype(jnp.int32).T
                 .reshape(2, nb, _B).transpose(1, 0, 2).reshape(-1))
    return _lj_launch(pairs_lin, tab, consts, nb // _CB)
